# baseline (speedup 1.0000x reference)
import functools

import jax
import jax.numpy as jnp
from jax import lax
from jax.experimental import pallas as pl
from jax.experimental.pallas import tpu as pltpu
from jax.experimental.pallas import tpu_sc as plsc

N = 10000
E = 160000
D = 256
DH = 128
NC = 2
NS = 16
K = 64
CH = 160
NB = 4
SEG = 32
EP = NS * CH * K
RPS = 632
NPAD = NS * RPS

_mesh = plsc.VectorSubcoreMesh(core_axis_name="c", subcore_axis_name="s")


@functools.partial(
    pl.kernel,
    mesh=_mesh,
    compiler_params=pltpu.CompilerParams(use_tc_tiling_on_sc=False),
    out_type=[
        jax.ShapeDtypeStruct((NC, NPAD, DH), jnp.float32),
        jax.ShapeDtypeStruct((NC, NPAD, 16), jnp.float32),
    ],
    scratch_types=[
        pltpu.VMEM((SEG, K), jnp.int32),
        pltpu.VMEM((SEG, K), jnp.int32),
        pltpu.VMEM((K, DH), jnp.float32),
        pltpu.VMEM((K, DH), jnp.float32),
        pltpu.VMEM((K, DH), jnp.float32),
        pltpu.VMEM((K, DH), jnp.float32),
        pltpu.VMEM((K, 16), jnp.float32),
        pltpu.VMEM_SHARED((NPAD, DH), jnp.float32),
        pltpu.VMEM_SHARED((NPAD, 16), jnp.float32),
        pltpu.SemaphoreType.DMA,
        pltpu.SemaphoreType.DMA,
        pltpu.SemaphoreType.DMA,
        pltpu.SemaphoreType.DMA,
        pltpu.SemaphoreType.DMA,
        pltpu.SemaphoreType.DMA,
        pltpu.SemaphoreType.DMA,
        pltpu.SemaphoreType.DMA,
        pltpu.SemaphoreType.DMA,
        pltpu.SemaphoreType.DMA,
        pltpu.SemaphoreType.DMA,
        pltpu.SemaphoreType.DMA,
    ],
)
def _sc_aggregate(xcat, rowp, colp, zacc, zcnt, ones, out, cntout,
                  rbuf, cbuf, gbuf0, gbuf1, gbuf2, gbuf3, obuf, acc, cnt,
                  gsem0, gsem1, gsem2, gsem3,
                  asem0, asem1, asem2, asem3,
                  csem0, csem1, csem2, csem3):
    c = lax.axis_index("c")
    s = lax.axis_index("s")
    base = s * RPS
    ebase = (c * NS + s) * CH
    cbase = s * CH

    gbufs = [gbuf0, gbuf1, gbuf2, gbuf3]
    gsems = [gsem0, gsem1, gsem2, gsem3]
    asems = [asem0, asem1, asem2, asem3]
    csems = [csem0, csem1, csem2, csem3]

    pltpu.sync_copy(zacc, acc.at[pl.ds(base, RPS)])
    pltpu.sync_copy(zcnt, cnt.at[pl.ds(base, RPS)])
    pltpu.sync_copy(ones, obuf)
    plsc.subcore_barrier()

    def fire_gather(j, gb, sem):
        pltpu.async_copy(xcat.at[rbuf.at[j]], gb, sem)

    def wait_gather(gb, sem):
        pltpu.make_async_copy(xcat.at[rbuf.at[0]], gb, sem).wait()

    def fire_scatter(j, gb, asem, csem, do_cnt):
        pltpu.async_copy(gb, acc.at[cbuf.at[j]], asem, add=True)

        @pl.when(do_cnt)
        def _():
            pltpu.async_copy(obuf, cnt.at[cbuf.at[j]], csem, add=True)

    def wait_scatter(gb, asem, csem, do_cnt):
        pltpu.make_async_copy(gb, acc.at[cbuf.at[0]], asem).wait()

        @pl.when(do_cnt)
        def _():
            pltpu.make_async_copy(obuf, cnt.at[cbuf.at[0]], csem).wait()

    @pl.loop(0, CH, step=SEG)
    def _(g0):
        pltpu.sync_copy(rowp.at[pl.ds(ebase + g0, SEG)], rbuf)
        pltpu.sync_copy(colp.at[pl.ds(cbase + g0, SEG)], cbuf)
        for b in range(NB):
            fire_gather(b, gbufs[b], gsems[b])

        @pl.loop(0, SEG, step=NB)
        def _(j):
            for b in range(NB):
                do_cnt = (b % NC) == c
                wait_gather(gbufs[b], gsems[b])
                fire_scatter(j + b, gbufs[b], asems[b], csems[b], do_cnt)

            for b in range(NB):
                do_cnt = (b % NC) == c
                wait_scatter(gbufs[b], asems[b], csems[b], do_cnt)

                @pl.when(j + NB + b < SEG)
                def _():
                    fire_gather(j + NB + b, gbufs[b], gsems[b])

    plsc.subcore_barrier()

    pltpu.sync_copy(acc.at[pl.ds(base, RPS)], out.at[c, pl.ds(base, RPS)])
    pltpu.sync_copy(cnt.at[pl.ds(base, RPS)], cntout.at[c, pl.ds(base, RPS)])


def _tc_body(sum_ref, x_ref, c0_ref, c1_ref, wl_ref, wr_ref, b_ref, o_ref):
    count = c0_ref[...][:, 0:1] + c1_ref[...][:, 0:1]
    rec = 1.0 / jnp.maximum(count, 1.0)
    outl = jnp.dot(sum_ref[...], wl_ref[...],
                   preferred_element_type=jnp.float32,
                   precision=lax.Precision.HIGHEST)
    outr = jnp.dot(x_ref[...], wr_ref[...],
                   preferred_element_type=jnp.float32,
                   precision=lax.Precision.HIGHEST)
    o_ref[...] = outl * rec + outr + b_ref[...]


def _tc_linear(summed, x, cnt0, cnt1, wlT, wrT, bias):
    blk = 1000
    return pl.pallas_call(
        _tc_body,
        grid=(N // blk,),
        in_specs=[
            pl.BlockSpec((blk, D), lambda i: (i, 0)),
            pl.BlockSpec((blk, D), lambda i: (i, 0)),
            pl.BlockSpec((blk, 16), lambda i: (i, 0)),
            pl.BlockSpec((blk, 16), lambda i: (i, 0)),
            pl.BlockSpec((D, D), lambda i: (0, 0)),
            pl.BlockSpec((D, D), lambda i: (0, 0)),
            pl.BlockSpec((1, D), lambda i: (0, 0)),
        ],
        out_specs=pl.BlockSpec((blk, D), lambda i: (i, 0)),
        out_shape=jax.ShapeDtypeStruct((N, D), jnp.float32),
    )(summed, x, cnt0, cnt1, wlT, wrT, bias)


@jax.jit
def kernel(x, edge_index, W_l, b_l, W_r, b_r):
    row = edge_index[0].astype(jnp.int32)
    col = edge_index[1].astype(jnp.int32)

    xcat = jnp.concatenate([x[:, :DH], x[:, DH:]], axis=0)

    pad = EP - E
    rowp = jnp.pad(row, (0, pad))
    colp = jnp.pad(col, (0, pad), constant_values=N)
    rowp2 = jnp.stack([rowp, rowp + N]).reshape(NC * NS * CH, K)
    colp2 = colp.reshape(NS * CH, K)

    zacc = jnp.zeros((RPS, DH), jnp.float32)
    zcnt = jnp.zeros((RPS, 16), jnp.float32)
    ones = jnp.ones((K, 16), jnp.float32)

    summ2, cnt = _sc_aggregate(xcat, rowp2, colp2, zacc, zcnt, ones)
    summed = jnp.concatenate([summ2[0, :N], summ2[1, :N]], axis=1)

    return _tc_linear(summed, x, cnt[0, :N], cnt[1, :N], W_l.T, W_r.T,
                      (b_l + b_r).reshape(1, D))


# speedup vs baseline: 1.0440x; 1.0440x over previous
import functools

import jax
import jax.numpy as jnp
from jax import lax
from jax.experimental import pallas as pl
from jax.experimental.pallas import tpu as pltpu
from jax.experimental.pallas import tpu_sc as plsc

N = 10000
E = 160000
D = 256
DH = 128
NC = 2
NS = 16
K = 128
CH = 80
SEG = 16
EP = NS * CH * K
RPS = 632
NPAD = NS * RPS

_mesh = plsc.VectorSubcoreMesh(core_axis_name="c", subcore_axis_name="s")


@functools.partial(
    pl.kernel,
    mesh=_mesh,
    compiler_params=pltpu.CompilerParams(use_tc_tiling_on_sc=False),
    out_type=[
        jax.ShapeDtypeStruct((NC, NPAD, DH), jnp.float32),
        jax.ShapeDtypeStruct((NPAD, 16), jnp.float32),
    ],
    scratch_types=[
        pltpu.VMEM((SEG, K), jnp.int32),
        pltpu.VMEM((SEG, K), jnp.int32),
        pltpu.VMEM((K, DH), jnp.float32),
        pltpu.VMEM((K, DH), jnp.float32),
        pltpu.VMEM((K, 16), jnp.float32),
        pltpu.VMEM_SHARED((NPAD, DH), jnp.float32),
        pltpu.VMEM_SHARED((NPAD, 16), jnp.float32),
        pltpu.SemaphoreType.DMA,
        pltpu.SemaphoreType.DMA,
        pltpu.SemaphoreType.DMA,
        pltpu.SemaphoreType.DMA,
        pltpu.SemaphoreType.DMA,
        pltpu.SemaphoreType.DMA,
    ],
)
def _sc_aggregate(xcat, rowp, colp, zacc, zcnt, ones, out, cntout,
                  rbuf, cbuf, gbuf0, gbuf1, obuf, acc, cnt,
                  gsem0, gsem1, asem0, asem1, csem0, csem1):
    c = lax.axis_index("c")
    s = lax.axis_index("s")
    base = s * RPS
    ebase = (c * NS + s) * CH
    cbase = s * CH

    pltpu.sync_copy(zacc, acc.at[pl.ds(base, RPS)])
    pltpu.sync_copy(zcnt, cnt.at[pl.ds(base, RPS)])
    pltpu.sync_copy(ones, obuf)
    plsc.subcore_barrier()

    def fire_gather(j, gb, sem):
        pltpu.async_copy(xcat.at[rbuf.at[j]], gb, sem)

    def wait_gather(gb, sem):
        pltpu.make_async_copy(xcat.at[rbuf.at[0]], gb, sem).wait()

    def fire_scatter(j, gb, asem, csem):
        pltpu.async_copy(gb, acc.at[cbuf.at[j]], asem, add=True)
        pltpu.async_copy(obuf, cnt.at[cbuf.at[j]], csem, add=True)

    def wait_scatter(gb, asem, csem):
        pltpu.make_async_copy(gb, acc.at[cbuf.at[0]], asem).wait()
        pltpu.make_async_copy(obuf, cnt.at[cbuf.at[0]], csem).wait()

    @pl.loop(0, CH, step=SEG)
    def _(g0):
        pltpu.sync_copy(rowp.at[pl.ds(ebase + g0, SEG)], rbuf)
        pltpu.sync_copy(colp.at[pl.ds(cbase + g0, SEG)], cbuf)
        fire_gather(0, gbuf0, gsem0)
        fire_gather(1, gbuf1, gsem1)

        @pl.loop(0, SEG, step=2)
        def _(j):
            wait_gather(gbuf0, gsem0)
            fire_scatter(j, gbuf0, asem0, csem0)
            wait_gather(gbuf1, gsem1)
            fire_scatter(j + 1, gbuf1, asem1, csem1)
            wait_scatter(gbuf0, asem0, csem0)

            @pl.when(j + 2 < SEG)
            def _():
                fire_gather(j + 2, gbuf0, gsem0)

            wait_scatter(gbuf1, asem1, csem1)

            @pl.when(j + 3 < SEG)
            def _():
                fire_gather(j + 3, gbuf1, gsem1)

    plsc.subcore_barrier()

    pltpu.sync_copy(acc.at[pl.ds(base, RPS)], out.at[c, pl.ds(base, RPS)])

    @pl.when(c == 0)
    def _():
        pltpu.sync_copy(cnt.at[pl.ds(base, RPS)], cntout.at[pl.ds(base, RPS)])


def _tc_body(s0_ref, s1_ref, x_ref, c_ref, wl0_ref, wl1_ref, wr_ref, b_ref,
             o_ref):
    rec = 1.0 / jnp.maximum(c_ref[...][:, 0:1], 1.0)
    outl = jnp.dot(s0_ref[0], wl0_ref[...],
                   preferred_element_type=jnp.float32,
                   precision=lax.Precision.HIGHEST)
    outl += jnp.dot(s1_ref[0], wl1_ref[...],
                    preferred_element_type=jnp.float32,
                    precision=lax.Precision.HIGHEST)
    outr = jnp.dot(x_ref[...], wr_ref[...],
                   preferred_element_type=jnp.float32,
                   precision=lax.Precision.HIGHEST)
    o_ref[...] = outl * rec + outr + b_ref[...]


def _tc_linear(summ2, x, cnt, wlT, wrT, bias):
    blk = 1000
    return pl.pallas_call(
        _tc_body,
        grid=(N // blk,),
        in_specs=[
            pl.BlockSpec((1, blk, DH), lambda i: (0, i, 0)),
            pl.BlockSpec((1, blk, DH), lambda i: (1, i, 0)),
            pl.BlockSpec((blk, D), lambda i: (i, 0)),
            pl.BlockSpec((blk, 16), lambda i: (i, 0)),
            pl.BlockSpec((DH, D), lambda i: (0, 0)),
            pl.BlockSpec((DH, D), lambda i: (1, 0)),
            pl.BlockSpec((D, D), lambda i: (0, 0)),
            pl.BlockSpec((1, D), lambda i: (0, 0)),
        ],
        out_specs=pl.BlockSpec((blk, D), lambda i: (i, 0)),
        out_shape=jax.ShapeDtypeStruct((N, D), jnp.float32),
    )(summ2, summ2, x, cnt, wlT, wlT, wrT, bias)


@jax.jit
def kernel(x, edge_index, W_l, b_l, W_r, b_r):
    row = edge_index[0].astype(jnp.int32)
    col = edge_index[1].astype(jnp.int32)

    xcat = jnp.concatenate([x[:, :DH], x[:, DH:]], axis=0)

    pad = EP - E
    rowp = jnp.pad(row, (0, pad))
    colp = jnp.pad(col, (0, pad), constant_values=N)
    rowp2 = jnp.stack([rowp, rowp + N]).reshape(NC * NS * CH, K)
    colp2 = colp.reshape(NS * CH, K)

    zacc = jnp.zeros((RPS, DH), jnp.float32)
    zcnt = jnp.zeros((RPS, 16), jnp.float32)
    ones = jnp.ones((K, 16), jnp.float32)

    summ2, cnt = _sc_aggregate(xcat, rowp2, colp2, zacc, zcnt, ones)

    return _tc_linear(summ2, x, cnt, W_l.T, W_r.T,
                      (b_l + b_r).reshape(1, D))


# speedup vs baseline: 1.0516x; 1.0073x over previous
import functools

import jax
import jax.numpy as jnp
from jax import lax
from jax.experimental import pallas as pl
from jax.experimental.pallas import tpu as pltpu
from jax.experimental.pallas import tpu_sc as plsc

N = 10000
E = 160000
D = 256
DH = 128
NC = 2
NS = 16
K = 128
CH = 80
SEG = 10
NSEGS = CH // SEG
EP = NS * CH * K
RPS = 632
NPAD = NS * RPS

_mesh = plsc.VectorSubcoreMesh(core_axis_name="c", subcore_axis_name="s")


@functools.partial(
    pl.kernel,
    mesh=_mesh,
    compiler_params=pltpu.CompilerParams(use_tc_tiling_on_sc=False),
    out_type=[
        jax.ShapeDtypeStruct((NC, NPAD, DH), jnp.float32),
        jax.ShapeDtypeStruct((NPAD, 16), jnp.float32),
    ],
    scratch_types=[
        pltpu.VMEM((2, SEG, K), jnp.int32),
        pltpu.VMEM((2, SEG, K), jnp.int32),
        pltpu.VMEM((K, DH), jnp.float32),
        pltpu.VMEM((K, DH), jnp.float32),
        pltpu.VMEM((K, 16), jnp.float32),
        pltpu.VMEM_SHARED((NPAD, DH), jnp.float32),
        pltpu.VMEM_SHARED((NPAD, 16), jnp.float32),
        pltpu.SemaphoreType.DMA,
        pltpu.SemaphoreType.DMA,
        pltpu.SemaphoreType.DMA,
        pltpu.SemaphoreType.DMA,
        pltpu.SemaphoreType.DMA,
        pltpu.SemaphoreType.DMA,
        pltpu.SemaphoreType.DMA,
    ],
)
def _sc_aggregate(xcat, rowp, colp, zacc, zcnt, ones, out, cntout,
                  rbuf, cbuf, gbuf0, gbuf1, obuf, acc, cnt,
                  gsem0, gsem1, asem0, asem1, csem0, csem1, isem):
    c = lax.axis_index("c")
    s = lax.axis_index("s")
    base = s * RPS
    ebase = (c * NS + s) * CH
    cbase = s * CH

    def ridx(ch):
        return rbuf.at[(ch // SEG) % 2].at[ch % SEG]

    def cidx(ch):
        return cbuf.at[(ch // SEG) % 2].at[ch % SEG]

    def fire_gather(ch, gb, sem):
        pltpu.async_copy(xcat.at[ridx(ch)], gb, sem)

    def wait_gather(gb, sem):
        pltpu.make_async_copy(xcat.at[rbuf.at[0].at[0]], gb, sem).wait()

    def fire_scatter(ch, gb, asem, csem):
        pltpu.async_copy(gb, acc.at[cidx(ch)], asem, add=True)
        pltpu.async_copy(obuf, cnt.at[cidx(ch)], csem, add=True)

    def wait_scatter(gb, asem, csem):
        pltpu.make_async_copy(gb, acc.at[cbuf.at[0].at[0]], asem).wait()
        pltpu.make_async_copy(obuf, cnt.at[cbuf.at[0].at[0]], csem).wait()

    def fire_idx_load(sg):
        h = sg % 2
        pltpu.async_copy(rowp.at[pl.ds(ebase + sg * SEG, SEG)],
                         rbuf.at[h], isem)
        pltpu.async_copy(colp.at[pl.ds(cbase + sg * SEG, SEG)],
                         cbuf.at[h], isem)

    def wait_idx_load():
        pltpu.make_async_copy(rowp.at[pl.ds(0, SEG)], rbuf.at[0], isem).wait()
        pltpu.make_async_copy(colp.at[pl.ds(0, SEG)], cbuf.at[0], isem).wait()

    pltpu.sync_copy(rowp.at[pl.ds(ebase, SEG)], rbuf.at[0])
    pltpu.sync_copy(colp.at[pl.ds(cbase, SEG)], cbuf.at[0])
    fire_gather(0, gbuf0, gsem0)
    fire_gather(1, gbuf1, gsem1)
    pltpu.sync_copy(ones, obuf)
    pltpu.sync_copy(zacc, acc.at[pl.ds(base, RPS)])
    pltpu.sync_copy(zcnt, cnt.at[pl.ds(base, RPS)])
    plsc.subcore_barrier()

    @pl.loop(0, CH, step=2)
    def _(j):
        @pl.when(j % SEG == 0)
        def _():
            @pl.when(j // SEG + 1 < NSEGS)
            def _():
                fire_idx_load(j // SEG + 1)

        wait_gather(gbuf0, gsem0)
        fire_scatter(j, gbuf0, asem0, csem0)
        wait_gather(gbuf1, gsem1)
        fire_scatter(j + 1, gbuf1, asem1, csem1)
        wait_scatter(gbuf0, asem0, csem0)

        @pl.when(j + 2 < CH)
        def _():
            @pl.when((j + 2) % SEG == 0)
            def _():
                wait_idx_load()

            fire_gather(j + 2, gbuf0, gsem0)

        wait_scatter(gbuf1, asem1, csem1)

        @pl.when(j + 3 < CH)
        def _():
            fire_gather(j + 3, gbuf1, gsem1)

    plsc.subcore_barrier()

    pltpu.sync_copy(acc.at[pl.ds(base, RPS)], out.at[c, pl.ds(base, RPS)])

    @pl.when(c == 0)
    def _():
        pltpu.sync_copy(cnt.at[pl.ds(base, RPS)], cntout.at[pl.ds(base, RPS)])


def _tc_body(sum_ref, x_ref, c_ref, wl_ref, wr_ref, b_ref, o_ref):
    rec = 1.0 / jnp.maximum(c_ref[...][:, 0:1], 1.0)
    outl = jnp.dot(sum_ref[...], wl_ref[...],
                   preferred_element_type=jnp.float32,
                   precision=lax.Precision.HIGHEST)
    outr = jnp.dot(x_ref[...], wr_ref[...],
                   preferred_element_type=jnp.float32,
                   precision=lax.Precision.HIGHEST)
    o_ref[...] = outl * rec + outr + b_ref[...]


def _tc_linear(summed, x, cnt, wlT, wrT, bias):
    blk = 1000
    return pl.pallas_call(
        _tc_body,
        grid=(N // blk,),
        in_specs=[
            pl.BlockSpec((blk, D), lambda i: (i, 0)),
            pl.BlockSpec((blk, D), lambda i: (i, 0)),
            pl.BlockSpec((blk, 16), lambda i: (i, 0)),
            pl.BlockSpec((D, D), lambda i: (0, 0)),
            pl.BlockSpec((D, D), lambda i: (0, 0)),
            pl.BlockSpec((1, D), lambda i: (0, 0)),
        ],
        out_specs=pl.BlockSpec((blk, D), lambda i: (i, 0)),
        out_shape=jax.ShapeDtypeStruct((N, D), jnp.float32),
    )(summed, x, cnt, wlT, wrT, bias)


@jax.jit
def kernel(x, edge_index, W_l, b_l, W_r, b_r):
    row = edge_index[0].astype(jnp.int32)
    col = edge_index[1].astype(jnp.int32)

    xcat = jnp.concatenate([x[:, :DH], x[:, DH:]], axis=0)

    pad = EP - E
    rowp = jnp.pad(row, (0, pad))
    colp = jnp.pad(col, (0, pad), constant_values=N)
    rowp2 = jnp.stack([rowp, rowp + N]).reshape(NC * NS * CH, K)
    colp2 = colp.reshape(NS * CH, K)

    zacc = jnp.zeros((RPS, DH), jnp.float32)
    zcnt = jnp.zeros((RPS, 16), jnp.float32)
    ones = jnp.ones((K, 16), jnp.float32)

    summ2, cnt = _sc_aggregate(xcat, rowp2, colp2, zacc, zcnt, ones)
    summed = jnp.concatenate([summ2[0, :N], summ2[1, :N]], axis=1)

    return _tc_linear(summed, x, cnt[:N], W_l.T, W_r.T,
                      (b_l + b_r).reshape(1, D))


# speedup vs baseline: 1.0619x; 1.0098x over previous
import functools

import jax
import jax.numpy as jnp
from jax import lax
from jax.experimental import pallas as pl
from jax.experimental.pallas import tpu as pltpu
from jax.experimental.pallas import tpu_sc as plsc

N = 10000
E = 160000
D = 256
DH = 128
NC = 2
NS = 16
K = 128
CH = 80
SEG = 16
EP = NS * CH * K
RPS = 632
NPAD = NS * RPS

_mesh = plsc.VectorSubcoreMesh(core_axis_name="c", subcore_axis_name="s")


@functools.partial(
    pl.kernel,
    mesh=_mesh,
    compiler_params=pltpu.CompilerParams(use_tc_tiling_on_sc=False),
    out_type=[
        jax.ShapeDtypeStruct((NC, NPAD, DH), jnp.float32),
        jax.ShapeDtypeStruct((NPAD, 16), jnp.float32),
    ],
    scratch_types=[
        pltpu.VMEM((SEG, K), jnp.int32),
        pltpu.VMEM((SEG, K), jnp.int32),
        pltpu.VMEM((K, DH), jnp.float32),
        pltpu.VMEM((K, DH), jnp.float32),
        pltpu.VMEM((K, 16), jnp.float32),
        pltpu.VMEM_SHARED((NPAD, DH), jnp.float32),
        pltpu.VMEM_SHARED((NPAD, 16), jnp.float32),
        pltpu.SemaphoreType.DMA,
        pltpu.SemaphoreType.DMA,
        pltpu.SemaphoreType.DMA,
        pltpu.SemaphoreType.DMA,
        pltpu.SemaphoreType.DMA,
        pltpu.SemaphoreType.DMA,
    ],
)
def _sc_aggregate(xcat, rowp, colp, zacc, zcnt, ones, out, cntout,
                  rbuf, cbuf, gbuf0, gbuf1, obuf, acc, cnt,
                  gsem0, gsem1, asem0, asem1, csem0, csem1):
    c = lax.axis_index("c")
    s = lax.axis_index("s")
    base = s * RPS
    ebase = (c * NS + s) * CH
    cbase = s * CH

    pltpu.sync_copy(zacc, acc.at[pl.ds(base, RPS)])
    pltpu.sync_copy(zcnt, cnt.at[pl.ds(base, RPS)])
    pltpu.sync_copy(ones, obuf)
    plsc.subcore_barrier()

    def fire_gather(j, gb, sem):
        pltpu.async_copy(xcat.at[rbuf.at[j]], gb, sem)

    def wait_gather(gb, sem):
        pltpu.make_async_copy(xcat.at[rbuf.at[0]], gb, sem).wait()

    def fire_scatter(j, gb, asem, csem):
        pltpu.async_copy(gb, acc.at[cbuf.at[j]], asem, add=True)
        pltpu.async_copy(obuf, cnt.at[cbuf.at[j]], csem, add=True)

    def wait_scatter(gb, asem, csem):
        pltpu.make_async_copy(gb, acc.at[cbuf.at[0]], asem).wait()
        pltpu.make_async_copy(obuf, cnt.at[cbuf.at[0]], csem).wait()

    @pl.loop(0, CH, step=SEG)
    def _(g0):
        pltpu.sync_copy(rowp.at[pl.ds(ebase + g0, SEG)], rbuf)
        pltpu.sync_copy(colp.at[pl.ds(cbase + g0, SEG)], cbuf)
        fire_gather(0, gbuf0, gsem0)
        fire_gather(1, gbuf1, gsem1)

        @pl.loop(0, SEG, step=2)
        def _(j):
            wait_gather(gbuf0, gsem0)
            fire_scatter(j, gbuf0, asem0, csem0)
            wait_gather(gbuf1, gsem1)
            fire_scatter(j + 1, gbuf1, asem1, csem1)
            wait_scatter(gbuf0, asem0, csem0)

            @pl.when(j + 2 < SEG)
            def _():
                fire_gather(j + 2, gbuf0, gsem0)

            wait_scatter(gbuf1, asem1, csem1)

            @pl.when(j + 3 < SEG)
            def _():
                fire_gather(j + 3, gbuf1, gsem1)

    plsc.subcore_barrier()

    pltpu.sync_copy(acc.at[pl.ds(base, RPS)], out.at[c, pl.ds(base, RPS)])

    @pl.when(c == 0)
    def _():
        pltpu.sync_copy(cnt.at[pl.ds(base, RPS)], cntout.at[pl.ds(base, RPS)])


def _tc_body(sum_ref, x_ref, c_ref, wl_ref, wr_ref, b_ref, o_ref):
    rec = 1.0 / jnp.maximum(c_ref[...][:, 0:1], 1.0)
    outl = jnp.dot(sum_ref[...], wl_ref[...],
                   preferred_element_type=jnp.float32,
                   precision=lax.Precision.HIGHEST)
    outr = jnp.dot(x_ref[...], wr_ref[...],
                   preferred_element_type=jnp.float32,
                   precision=lax.Precision.HIGHEST)
    o_ref[...] = outl * rec + outr + b_ref[...]


def _tc_linear(summed, x, cnt, wlT, wrT, bias):
    blk = 1000
    return pl.pallas_call(
        _tc_body,
        grid=(N // blk,),
        in_specs=[
            pl.BlockSpec((blk, D), lambda i: (i, 0)),
            pl.BlockSpec((blk, D), lambda i: (i, 0)),
            pl.BlockSpec((blk, 16), lambda i: (i, 0)),
            pl.BlockSpec((D, D), lambda i: (0, 0)),
            pl.BlockSpec((D, D), lambda i: (0, 0)),
            pl.BlockSpec((1, D), lambda i: (0, 0)),
        ],
        out_specs=pl.BlockSpec((blk, D), lambda i: (i, 0)),
        out_shape=jax.ShapeDtypeStruct((N, D), jnp.float32),
    )(summed, x, cnt, wlT, wrT, bias)


@jax.jit
def kernel(x, edge_index, W_l, b_l, W_r, b_r):
    row = edge_index[0].astype(jnp.int32)
    col = edge_index[1].astype(jnp.int32)

    xcat = jnp.concatenate([x[:, :DH], x[:, DH:]], axis=0)

    pad = EP - E
    rowp = jnp.pad(row, (0, pad))
    colp = jnp.pad(col, (0, pad), constant_values=N)
    rowp2 = jnp.stack([rowp, rowp + N]).reshape(NC * NS * CH, K)
    colp2 = colp.reshape(NS * CH, K)

    zacc = jnp.zeros((RPS, DH), jnp.float32)
    zcnt = jnp.zeros((RPS, 16), jnp.float32)
    ones = jnp.ones((K, 16), jnp.float32)

    summ2, cnt = _sc_aggregate(xcat, rowp2, colp2, zacc, zcnt, ones)
    summed = jnp.concatenate([summ2[0, :N], summ2[1, :N]], axis=1)

    return _tc_linear(summed, x, cnt[:N], W_l.T, W_r.T,
                      (b_l + b_r).reshape(1, D))
